# baseline (speedup 1.0000x reference)
import functools
import math

import jax
import jax.numpy as jnp
from jax import lax
from jax.experimental import pallas as pl
from jax.experimental.pallas import tpu as pltpu
from jax.experimental.pallas import tpu_sc as plsc

N = 10000
E = 320000
D = 128
DH = D // 2
G = 64
C = 10

NP = 10240
NSC = 16
CHUNK = 128
NCH = 160
NPH = 4
CPP = NCH // NPH
EP = NSC * NCH * CHUNK
ROWS_PER_SUB = NP // NSC
TAB_PER_SUB = NP // NSC
ZR = 16
NZB = ROWS_PER_SUB // ZR


def _sc_agg_body(ha_hbm, hb_hbm, srcs_hbm, dsts_hbm, out_hbm,
                 src_v, dst_v, rows_v, rows1_v, rows2_v, rows3_v, zero_v,
                 htab_sh, acc_sh, sem0, sem1, sem2, sem3):
    c = lax.axis_index("c")
    s = lax.axis_index("s")

    @pl.when(c == 0)
    def _():
        pltpu.sync_copy(ha_hbm.at[pl.ds(s * TAB_PER_SUB, TAB_PER_SUB)],
                        htab_sh.at[pl.ds(s * TAB_PER_SUB, TAB_PER_SUB)])

    @pl.when(c == 1)
    def _():
        pltpu.sync_copy(hb_hbm.at[pl.ds(s * TAB_PER_SUB, TAB_PER_SUB)],
                        htab_sh.at[pl.ds(s * TAB_PER_SUB, TAB_PER_SUB)])

    def zrow(i, carry):
        for l in range(DH // 16):
            zero_v[i, pl.ds(l * 16, 16)] = jnp.zeros((16,), jnp.float32)
        return carry
    lax.fori_loop(0, ZR, zrow, 0)

    def zblk(j, carry):
        pltpu.sync_copy(zero_v, acc_sh.at[pl.ds(s * ROWS_PER_SUB + j * ZR, ZR)])
        return carry
    lax.fori_loop(0, NZB, zblk, 0)

    plsc.subcore_barrier()

    for p in range(NPH):
        pltpu.sync_copy(srcs_hbm.at[s].at[pl.ds(p * CPP, CPP)], src_v)
        pltpu.sync_copy(dsts_hbm.at[s].at[pl.ds(p * CPP, CPP)], dst_v)

        bufs = (rows_v, rows1_v, rows2_v, rows3_v)
        sems = (sem0, sem1, sem2, sem3)
        for b in range(4):
            pltpu.async_copy(htab_sh.at[src_v.at[b]], bufs[b], sems[b])

        def body(jj, carry):
            j = 4 * jj
            for b in range(4):
                pltpu.make_async_copy(
                    htab_sh.at[src_v.at[j + b]], bufs[b], sems[b]).wait()
                pltpu.sync_copy(bufs[b], acc_sh.at[dst_v.at[j + b]], add=True)

                @pl.when(jj < CPP // 4 - 1)
                def _():
                    pltpu.async_copy(
                        htab_sh.at[src_v.at[j + b + 4]], bufs[b], sems[b])
            return carry
        lax.fori_loop(0, CPP // 4, body, 0)
    plsc.subcore_barrier()

    pltpu.sync_copy(acc_sh.at[pl.ds(s * ROWS_PER_SUB, ROWS_PER_SUB)],
                    out_hbm.at[c].at[pl.ds(s * ROWS_PER_SUB, ROWS_PER_SUB)])


@functools.cache
def _sc_agg_call():
    return pl.kernel(
        _sc_agg_body,
        out_type=jax.ShapeDtypeStruct((2, NP, DH), jnp.float32),
        mesh=plsc.VectorSubcoreMesh(core_axis_name="c", subcore_axis_name="s"),
        scratch_types=[
            pltpu.VMEM((CPP, CHUNK), jnp.int32),
            pltpu.VMEM((CPP, CHUNK), jnp.int32),
            pltpu.VMEM((CHUNK, DH), jnp.float32),
            pltpu.VMEM((CHUNK, DH), jnp.float32),
            pltpu.VMEM((CHUNK, DH), jnp.float32),
            pltpu.VMEM((CHUNK, DH), jnp.float32),
            pltpu.VMEM((ZR, DH), jnp.float32),
            pltpu.VMEM_SHARED((NP, DH), jnp.float32),
            pltpu.VMEM_SHARED((NP, DH), jnp.float32),
            pltpu.SemaphoreType.DMA,
            pltpu.SemaphoreType.DMA,
            pltpu.SemaphoreType.DMA,
            pltpu.SemaphoreType.DMA,
        ],
        compiler_params=pltpu.CompilerParams(use_tc_tiling_on_sc=False),
    )


def _sc_agg(h2, src, dst):
    return _sc_agg_call()(h2[0], h2[1], src, dst)


BM = 512


def _bn_body(x_ref, g_ref, b_ref, o_ref):
    y = x_ref[...] * g_ref[...] + b_ref[...]
    o_ref[0] = y[:, :DH]
    o_ref[1] = y[:, DH:]


def _bn(x, gv, bv):
    return pl.pallas_call(
        _bn_body,
        grid=(NP // BM,),
        in_specs=[
            pl.BlockSpec((BM, D), lambda i: (i, 0)),
            pl.BlockSpec((1, D), lambda i: (0, 0)),
            pl.BlockSpec((1, D), lambda i: (0, 0)),
        ],
        out_specs=pl.BlockSpec((2, BM, DH), lambda i: (0, i, 0)),
        out_shape=jax.ShapeDtypeStruct((2, NP, DH), jnp.float32),
    )(x, gv, bv)


def _layer_body(a_ref, h_ref, wr_ref, wo_ref, b_ref, o_ref, *, relu):
    acc = jnp.dot(a_ref[0], wr_ref[0], preferred_element_type=jnp.float32)
    acc = acc + jnp.dot(a_ref[1], wr_ref[1], preferred_element_type=jnp.float32)
    acc = acc + jnp.dot(h_ref[0], wo_ref[0], preferred_element_type=jnp.float32)
    acc = acc + jnp.dot(h_ref[1], wo_ref[1], preferred_element_type=jnp.float32)
    acc = acc + b_ref[...]
    if relu:
        acc = jnp.maximum(acc, 0.0)
    o_ref[0] = acc[:, :DH]
    o_ref[1] = acc[:, DH:]


def _layer(a2, h2, wr2, wo2, b, relu):
    return pl.pallas_call(
        functools.partial(_layer_body, relu=relu),
        grid=(NP // BM,),
        in_specs=[
            pl.BlockSpec((2, BM, DH), lambda i: (0, i, 0)),
            pl.BlockSpec((2, BM, DH), lambda i: (0, i, 0)),
            pl.BlockSpec((2, DH, D), lambda i: (0, 0, 0)),
            pl.BlockSpec((2, DH, D), lambda i: (0, 0, 0)),
            pl.BlockSpec((1, D), lambda i: (0, 0)),
        ],
        out_specs=pl.BlockSpec((2, BM, DH), lambda i: (0, i, 0)),
        out_shape=jax.ShapeDtypeStruct((2, NP, DH), jnp.float32),
    )(a2, h2, wr2, wo2, b)


def _pool_body(h_ref, b_ref, wl_ref, bl_ref, o_ref, s0_ref, s1_ref, cnts_ref):
    i = pl.program_id(0)

    @pl.when(i == 0)
    def _():
        s0_ref[...] = jnp.zeros_like(s0_ref)
        s1_ref[...] = jnp.zeros_like(s1_ref)
        cnts_ref[...] = jnp.zeros_like(cnts_ref)

    seg = b_ref[...]
    onehot = jnp.where(
        seg[:, None] == lax.broadcasted_iota(jnp.int32, (1, G), 1),
        1.0, 0.0).astype(jnp.float32)
    dn = (((0,), (0,)), ((), ()))
    s0_ref[...] += lax.dot_general(onehot, h_ref[0], dn,
                                   preferred_element_type=jnp.float32)
    s1_ref[...] += lax.dot_general(onehot, h_ref[1], dn,
                                   preferred_element_type=jnp.float32)
    cnts_ref[...] += jnp.sum(onehot, axis=0)[:, None]

    @pl.when(i == pl.num_programs(0) - 1)
    def _():
        cnt = jnp.maximum(cnts_ref[...], 1.0)
        p0 = s0_ref[...] / cnt
        p1 = s1_ref[...] / cnt
        o_ref[...] = (jnp.dot(p0, wl_ref[0], preferred_element_type=jnp.float32)
                      + jnp.dot(p1, wl_ref[1], preferred_element_type=jnp.float32)
                      + bl_ref[...])


def _pool(h2, segs, wl2, bl):
    return pl.pallas_call(
        _pool_body,
        grid=(NP // BM,),
        in_specs=[
            pl.BlockSpec((2, BM, DH), lambda i: (0, i, 0)),
            pl.BlockSpec((BM,), lambda i: (i,)),
            pl.BlockSpec((2, DH, D), lambda i: (0, 0, 0)),
            pl.BlockSpec((1, D), lambda i: (0, 0)),
        ],
        out_specs=pl.BlockSpec((G, D), lambda i: (0, 0)),
        out_shape=jax.ShapeDtypeStruct((G, D), jnp.float32),
        scratch_shapes=[
            pltpu.VMEM((G, DH), jnp.float32),
            pltpu.VMEM((G, DH), jnp.float32),
            pltpu.VMEM((G, DH), jnp.float32),
        ],
    )(h2, segs, wl2, bl)


def _split_w(w):
    return jnp.stack([w[:DH], w[DH:]])


def kernel(x, edge_index, batch, bn_gamma, bn_beta,
           W1_rel, W1_root, b1, W2_rel, W2_root, b2,
           W3_rel, W3_root, b3, W_lin, b_lin):
    eps = 1e-5
    gv = (bn_gamma * (1.0 / math.sqrt(1.0 + eps)))[None, :]
    bv = bn_beta[None, :]

    xp = jnp.pad(x, ((0, NP - N), (0, 0)))
    src = jnp.pad(edge_index[0], (0, EP - E)).reshape(NSC, NCH, CHUNK)
    pad_dst = N + (jnp.arange(EP - E, dtype=jnp.int32) % (NP - N))
    dst = jnp.concatenate([edge_index[1], pad_dst]).reshape(NSC, NCH, CHUNK)
    segs = jnp.pad(batch, (0, NP - N), constant_values=G).astype(jnp.int32)
    wl2 = _split_w(jnp.pad(W_lin, ((0, 0), (0, D - C))))
    bl = jnp.pad(b_lin, (0, D - C))[None, :]

    h2 = _bn(xp, gv, bv)

    for (wr, wo, b, relu) in (
        (W1_rel, W1_root, b1, True),
        (W2_rel, W2_root, b2, True),
        (W3_rel, W3_root, b3, False),
    ):
        a2 = _sc_agg(h2, src, dst)
        h2 = _layer(a2, h2, _split_w(wr), _split_w(wo), b[None, :], relu)

    out = _pool(h2, segs, wl2, bl)
    return out[:, :C]


# speedup vs baseline: 1.0674x; 1.0674x over previous
import functools
import math

import jax
import jax.numpy as jnp
from jax import lax
from jax.experimental import pallas as pl
from jax.experimental.pallas import tpu as pltpu
from jax.experimental.pallas import tpu_sc as plsc

N = 10000
E = 320000
D = 128
DH = D // 2
G = 64
C = 10

NP = 10240
NSC = 16
CHUNK = 128
NCH = 160
NPH = 2
CPP = NCH // NPH
EP = NSC * NCH * CHUNK
ROWS_PER_SUB = NP // NSC
TAB_PER_SUB = NP // NSC
ZR = 16
NZB = ROWS_PER_SUB // ZR


def _sc_agg_body(ha_hbm, hb_hbm, srcs_hbm, dsts_hbm, out_hbm,
                 src_v, dst_v, rows_v, rows1_v, zero_v, htab_sh, acc_sh,
                 sem0, sem1):
    c = lax.axis_index("c")
    s = lax.axis_index("s")

    @pl.when(c == 0)
    def _():
        pltpu.sync_copy(ha_hbm.at[pl.ds(s * TAB_PER_SUB, TAB_PER_SUB)],
                        htab_sh.at[pl.ds(s * TAB_PER_SUB, TAB_PER_SUB)])

    @pl.when(c == 1)
    def _():
        pltpu.sync_copy(hb_hbm.at[pl.ds(s * TAB_PER_SUB, TAB_PER_SUB)],
                        htab_sh.at[pl.ds(s * TAB_PER_SUB, TAB_PER_SUB)])

    def zrow(i, carry):
        for l in range(DH // 16):
            zero_v[i, pl.ds(l * 16, 16)] = jnp.zeros((16,), jnp.float32)
        return carry
    lax.fori_loop(0, ZR, zrow, 0)

    def zblk(j, carry):
        pltpu.sync_copy(zero_v, acc_sh.at[pl.ds(s * ROWS_PER_SUB + j * ZR, ZR)])
        return carry
    lax.fori_loop(0, NZB, zblk, 0)

    plsc.subcore_barrier()

    for p in range(NPH):
        pltpu.sync_copy(srcs_hbm.at[s].at[pl.ds(p * CPP, CPP)], src_v)
        pltpu.sync_copy(dsts_hbm.at[s].at[pl.ds(p * CPP, CPP)], dst_v)

        pltpu.async_copy(htab_sh.at[src_v.at[0]], rows_v, sem0)

        def body(jj, carry):
            j = 2 * jj
            pltpu.async_copy(htab_sh.at[src_v.at[j + 1]], rows1_v, sem1)
            pltpu.make_async_copy(htab_sh.at[src_v.at[j]], rows_v, sem0).wait()
            pltpu.sync_copy(rows_v, acc_sh.at[dst_v.at[j]], add=True)

            @pl.when(jj < CPP // 2 - 1)
            def _():
                pltpu.async_copy(htab_sh.at[src_v.at[j + 2]], rows_v, sem0)

            pltpu.make_async_copy(htab_sh.at[src_v.at[j + 1]], rows1_v, sem1).wait()
            pltpu.sync_copy(rows1_v, acc_sh.at[dst_v.at[j + 1]], add=True)
            return carry
        lax.fori_loop(0, CPP // 2, body, 0)
    plsc.subcore_barrier()

    pltpu.sync_copy(acc_sh.at[pl.ds(s * ROWS_PER_SUB, ROWS_PER_SUB)],
                    out_hbm.at[c].at[pl.ds(s * ROWS_PER_SUB, ROWS_PER_SUB)])


@functools.cache
def _sc_agg_call():
    return pl.kernel(
        _sc_agg_body,
        out_type=jax.ShapeDtypeStruct((2, NP, DH), jnp.float32),
        mesh=plsc.VectorSubcoreMesh(core_axis_name="c", subcore_axis_name="s"),
        scratch_types=[
            pltpu.VMEM((CPP, CHUNK), jnp.int32),
            pltpu.VMEM((CPP, CHUNK), jnp.int32),
            pltpu.VMEM((CHUNK, DH), jnp.float32),
            pltpu.VMEM((CHUNK, DH), jnp.float32),
            pltpu.VMEM((ZR, DH), jnp.float32),
            pltpu.VMEM_SHARED((NP, DH), jnp.float32),
            pltpu.VMEM_SHARED((NP, DH), jnp.float32),
            pltpu.SemaphoreType.DMA,
            pltpu.SemaphoreType.DMA,
        ],
        compiler_params=pltpu.CompilerParams(use_tc_tiling_on_sc=False),
    )


def _sc_agg(h2, src, dst):
    return _sc_agg_call()(h2[0], h2[1], src, dst)


BM = 512


def _bn_body(x_ref, g_ref, b_ref, o_ref):
    y = x_ref[...] * g_ref[...] + b_ref[...]
    o_ref[0] = y[:, :DH]
    o_ref[1] = y[:, DH:]


def _bn(x, gv, bv):
    return pl.pallas_call(
        _bn_body,
        grid=(NP // BM,),
        in_specs=[
            pl.BlockSpec((BM, D), lambda i: (i, 0)),
            pl.BlockSpec((1, D), lambda i: (0, 0)),
            pl.BlockSpec((1, D), lambda i: (0, 0)),
        ],
        out_specs=pl.BlockSpec((2, BM, DH), lambda i: (0, i, 0)),
        out_shape=jax.ShapeDtypeStruct((2, NP, DH), jnp.float32),
    )(x, gv, bv)


def _layer_body(a_ref, h_ref, wr_ref, wo_ref, b_ref, o_ref, *, relu):
    acc = jnp.dot(a_ref[0], wr_ref[0], preferred_element_type=jnp.float32)
    acc = acc + jnp.dot(a_ref[1], wr_ref[1], preferred_element_type=jnp.float32)
    acc = acc + jnp.dot(h_ref[0], wo_ref[0], preferred_element_type=jnp.float32)
    acc = acc + jnp.dot(h_ref[1], wo_ref[1], preferred_element_type=jnp.float32)
    acc = acc + b_ref[...]
    if relu:
        acc = jnp.maximum(acc, 0.0)
    o_ref[0] = acc[:, :DH]
    o_ref[1] = acc[:, DH:]


def _layer(a2, h2, wr2, wo2, b, relu):
    return pl.pallas_call(
        functools.partial(_layer_body, relu=relu),
        grid=(NP // BM,),
        in_specs=[
            pl.BlockSpec((2, BM, DH), lambda i: (0, i, 0)),
            pl.BlockSpec((2, BM, DH), lambda i: (0, i, 0)),
            pl.BlockSpec((2, DH, D), lambda i: (0, 0, 0)),
            pl.BlockSpec((2, DH, D), lambda i: (0, 0, 0)),
            pl.BlockSpec((1, D), lambda i: (0, 0)),
        ],
        out_specs=pl.BlockSpec((2, BM, DH), lambda i: (0, i, 0)),
        out_shape=jax.ShapeDtypeStruct((2, NP, DH), jnp.float32),
    )(a2, h2, wr2, wo2, b)


def _layer3_pool_body(a_ref, h_ref, wr_ref, wo_ref, b_ref, seg_ref,
                      wl_ref, bl_ref, o_ref, sums_ref, cnts_ref):
    i = pl.program_id(0)

    @pl.when(i == 0)
    def _():
        sums_ref[...] = jnp.zeros_like(sums_ref)
        cnts_ref[...] = jnp.zeros_like(cnts_ref)

    acc = jnp.dot(a_ref[0], wr_ref[0], preferred_element_type=jnp.float32)
    acc = acc + jnp.dot(a_ref[1], wr_ref[1], preferred_element_type=jnp.float32)
    acc = acc + jnp.dot(h_ref[0], wo_ref[0], preferred_element_type=jnp.float32)
    acc = acc + jnp.dot(h_ref[1], wo_ref[1], preferred_element_type=jnp.float32)
    acc = acc + b_ref[...]

    seg = seg_ref[...]
    onehot = jnp.where(
        seg[:, None] == lax.broadcasted_iota(jnp.int32, (1, G), 1),
        1.0, 0.0).astype(jnp.float32)
    sums_ref[...] += lax.dot_general(onehot, acc, (((0,), (0,)), ((), ())),
                                     preferred_element_type=jnp.float32)
    cnts_ref[...] += jnp.sum(onehot, axis=0)[:, None]

    @pl.when(i == pl.num_programs(0) - 1)
    def _():
        pooled = sums_ref[...] / jnp.maximum(cnts_ref[...], 1.0)
        o_ref[...] = jnp.dot(pooled, wl_ref[...],
                             preferred_element_type=jnp.float32) + bl_ref[...]


def _layer3_pool(a2, h2, wr2, wo2, b, segs, wl, bl):
    return pl.pallas_call(
        _layer3_pool_body,
        grid=(NP // BM,),
        in_specs=[
            pl.BlockSpec((2, BM, DH), lambda i: (0, i, 0)),
            pl.BlockSpec((2, BM, DH), lambda i: (0, i, 0)),
            pl.BlockSpec((2, DH, D), lambda i: (0, 0, 0)),
            pl.BlockSpec((2, DH, D), lambda i: (0, 0, 0)),
            pl.BlockSpec((1, D), lambda i: (0, 0)),
            pl.BlockSpec((BM,), lambda i: (i,)),
            pl.BlockSpec((D, D), lambda i: (0, 0)),
            pl.BlockSpec((1, D), lambda i: (0, 0)),
        ],
        out_specs=pl.BlockSpec((G, D), lambda i: (0, 0)),
        out_shape=jax.ShapeDtypeStruct((G, D), jnp.float32),
        scratch_shapes=[
            pltpu.VMEM((G, D), jnp.float32),
            pltpu.VMEM((G, D), jnp.float32),
        ],
    )(a2, h2, wr2, wo2, b, segs, wl, bl)


def _split_w(w):
    return jnp.stack([w[:DH], w[DH:]])


def kernel(x, edge_index, batch, bn_gamma, bn_beta,
           W1_rel, W1_root, b1, W2_rel, W2_root, b2,
           W3_rel, W3_root, b3, W_lin, b_lin):
    eps = 1e-5
    gv = (bn_gamma * (1.0 / math.sqrt(1.0 + eps)))[None, :]
    bv = bn_beta[None, :]

    xp = jnp.pad(x, ((0, NP - N), (0, 0)))
    src = jnp.pad(edge_index[0], (0, EP - E)).reshape(NSC, NCH, CHUNK)
    pad_dst = N + (jnp.arange(EP - E, dtype=jnp.int32) % (NP - N))
    dst = jnp.concatenate([edge_index[1], pad_dst]).reshape(NSC, NCH, CHUNK)
    segs = jnp.pad(batch, (0, NP - N), constant_values=G).astype(jnp.int32)
    wl = jnp.pad(W_lin, ((0, 0), (0, D - C)))
    bl = jnp.pad(b_lin, (0, D - C))[None, :]

    h2 = _bn(xp, gv, bv)

    for (wr, wo, b, relu) in (
        (W1_rel, W1_root, b1, True),
        (W2_rel, W2_root, b2, True),
    ):
        a2 = _sc_agg(h2, src, dst)
        h2 = _layer(a2, h2, _split_w(wr), _split_w(wo), b[None, :], relu)

    a2 = _sc_agg(h2, src, dst)
    out = _layer3_pool(a2, h2, _split_w(W3_rel), _split_w(W3_root),
                       b3[None, :], segs, wl, bl)
    return out[:, :C]


# speedup vs baseline: 1.1097x; 1.0396x over previous
import functools
import math

import jax
import jax.numpy as jnp
from jax import lax
from jax.experimental import pallas as pl
from jax.experimental.pallas import tpu as pltpu
from jax.experimental.pallas import tpu_sc as plsc

N = 10000
E = 320000
D = 128
DH = D // 2
G = 64
C = 10

NP = 10240
NSC = 16
CHUNK = 128
NCH = 160
NPH = 2
CPP = NCH // NPH
EP = NSC * NCH * CHUNK
ROWS_PER_SUB = NP // NSC
TAB_PER_SUB = NP // NSC
ZR = 16
NZB = ROWS_PER_SUB // ZR


def _sc_agg_body(ha_hbm, hb_hbm, srcs_hbm, dsts_hbm, out_hbm,
                 src_v, dst_v, rows_v, rows1_v, zero_v, htab_sh, acc_sh,
                 sem0, sem1):
    c = lax.axis_index("c")
    s = lax.axis_index("s")

    @pl.when(c == 0)
    def _():
        pltpu.sync_copy(ha_hbm.at[pl.ds(s * TAB_PER_SUB, TAB_PER_SUB)],
                        htab_sh.at[pl.ds(s * TAB_PER_SUB, TAB_PER_SUB)])

    @pl.when(c == 1)
    def _():
        pltpu.sync_copy(hb_hbm.at[pl.ds(s * TAB_PER_SUB, TAB_PER_SUB)],
                        htab_sh.at[pl.ds(s * TAB_PER_SUB, TAB_PER_SUB)])

    def zrow(i, carry):
        for l in range(DH // 16):
            zero_v[i, pl.ds(l * 16, 16)] = jnp.zeros((16,), jnp.float32)
        return carry
    lax.fori_loop(0, ZR, zrow, 0)

    def zblk(j, carry):
        pltpu.sync_copy(zero_v, acc_sh.at[pl.ds(s * ROWS_PER_SUB + j * ZR, ZR)])
        return carry
    lax.fori_loop(0, NZB, zblk, 0)

    plsc.subcore_barrier()

    for p in range(NPH):
        pltpu.sync_copy(srcs_hbm.at[s].at[pl.ds(p * CPP, CPP)], src_v)
        pltpu.sync_copy(dsts_hbm.at[s].at[pl.ds(p * CPP, CPP)], dst_v)

        pltpu.async_copy(htab_sh.at[src_v.at[0]], rows_v, sem0)

        def body(jj, carry):
            j = 2 * jj
            pltpu.async_copy(htab_sh.at[src_v.at[j + 1]], rows1_v, sem1)
            pltpu.make_async_copy(htab_sh.at[src_v.at[j]], rows_v, sem0).wait()
            pltpu.sync_copy(rows_v, acc_sh.at[dst_v.at[j]], add=True)

            @pl.when(jj < CPP // 2 - 1)
            def _():
                pltpu.async_copy(htab_sh.at[src_v.at[j + 2]], rows_v, sem0)

            pltpu.make_async_copy(htab_sh.at[src_v.at[j + 1]], rows1_v, sem1).wait()
            pltpu.sync_copy(rows1_v, acc_sh.at[dst_v.at[j + 1]], add=True)
            return carry
        lax.fori_loop(0, CPP // 2, body, 0)
    plsc.subcore_barrier()

    pltpu.sync_copy(acc_sh.at[pl.ds(s * ROWS_PER_SUB, ROWS_PER_SUB)],
                    out_hbm.at[c].at[pl.ds(s * ROWS_PER_SUB, ROWS_PER_SUB)])


@functools.cache
def _sc_agg_call():
    return pl.kernel(
        _sc_agg_body,
        out_type=jax.ShapeDtypeStruct((2, NP, DH), jnp.float32),
        mesh=plsc.VectorSubcoreMesh(core_axis_name="c", subcore_axis_name="s"),
        scratch_types=[
            pltpu.VMEM((CPP, CHUNK), jnp.int32),
            pltpu.VMEM((CPP, CHUNK), jnp.int32),
            pltpu.VMEM((CHUNK, DH), jnp.float32),
            pltpu.VMEM((CHUNK, DH), jnp.float32),
            pltpu.VMEM((ZR, DH), jnp.float32),
            pltpu.VMEM_SHARED((NP, DH), jnp.float32),
            pltpu.VMEM_SHARED((NP, DH), jnp.float32),
            pltpu.SemaphoreType.DMA,
            pltpu.SemaphoreType.DMA,
        ],
        compiler_params=pltpu.CompilerParams(use_tc_tiling_on_sc=False),
    )


def _sc_agg(h2, src, dst):
    return _sc_agg_call()(h2[0], h2[1], src, dst)


BM = 1024


def _bn_body(x_ref, g_ref, b_ref, o_ref):
    y = x_ref[...] * g_ref[...] + b_ref[...]
    o_ref[0] = y[:, :DH]
    o_ref[1] = y[:, DH:]


def _bn(x, gv, bv):
    return pl.pallas_call(
        _bn_body,
        grid=(NP // BM,),
        in_specs=[
            pl.BlockSpec((BM, D), lambda i: (i, 0)),
            pl.BlockSpec((1, D), lambda i: (0, 0)),
            pl.BlockSpec((1, D), lambda i: (0, 0)),
        ],
        out_specs=pl.BlockSpec((2, BM, DH), lambda i: (0, i, 0)),
        out_shape=jax.ShapeDtypeStruct((2, NP, DH), jnp.float32),
    )(x, gv, bv)


def _layer_body(a_ref, h_ref, wr_ref, wo_ref, b_ref, o_ref, *, relu):
    acc = jnp.dot(a_ref[0], wr_ref[0], preferred_element_type=jnp.float32)
    acc = acc + jnp.dot(a_ref[1], wr_ref[1], preferred_element_type=jnp.float32)
    acc = acc + jnp.dot(h_ref[0], wo_ref[0], preferred_element_type=jnp.float32)
    acc = acc + jnp.dot(h_ref[1], wo_ref[1], preferred_element_type=jnp.float32)
    acc = acc + b_ref[...]
    if relu:
        acc = jnp.maximum(acc, 0.0)
    o_ref[0] = acc[:, :DH]
    o_ref[1] = acc[:, DH:]


def _layer(a2, h2, wr2, wo2, b, relu):
    return pl.pallas_call(
        functools.partial(_layer_body, relu=relu),
        grid=(NP // BM,),
        in_specs=[
            pl.BlockSpec((2, BM, DH), lambda i: (0, i, 0)),
            pl.BlockSpec((2, BM, DH), lambda i: (0, i, 0)),
            pl.BlockSpec((2, DH, D), lambda i: (0, 0, 0)),
            pl.BlockSpec((2, DH, D), lambda i: (0, 0, 0)),
            pl.BlockSpec((1, D), lambda i: (0, 0)),
        ],
        out_specs=pl.BlockSpec((2, BM, DH), lambda i: (0, i, 0)),
        out_shape=jax.ShapeDtypeStruct((2, NP, DH), jnp.float32),
    )(a2, h2, wr2, wo2, b)


def _layer3_pool_body(a_ref, h_ref, wr_ref, wo_ref, b_ref, seg_ref,
                      wl_ref, bl_ref, o_ref, sums_ref, cnts_ref):
    i = pl.program_id(0)

    @pl.when(i == 0)
    def _():
        sums_ref[...] = jnp.zeros_like(sums_ref)
        cnts_ref[...] = jnp.zeros_like(cnts_ref)

    acc = jnp.dot(a_ref[0], wr_ref[0], preferred_element_type=jnp.float32)
    acc = acc + jnp.dot(a_ref[1], wr_ref[1], preferred_element_type=jnp.float32)
    acc = acc + jnp.dot(h_ref[0], wo_ref[0], preferred_element_type=jnp.float32)
    acc = acc + jnp.dot(h_ref[1], wo_ref[1], preferred_element_type=jnp.float32)
    acc = acc + b_ref[...]

    seg = seg_ref[...]
    onehot = jnp.where(
        seg[:, None] == lax.broadcasted_iota(jnp.int32, (1, G), 1),
        1.0, 0.0).astype(jnp.float32)
    sums_ref[...] += lax.dot_general(onehot, acc, (((0,), (0,)), ((), ())),
                                     preferred_element_type=jnp.float32)
    cnts_ref[...] += jnp.sum(onehot, axis=0)[:, None]

    @pl.when(i == pl.num_programs(0) - 1)
    def _():
        pooled = sums_ref[...] / jnp.maximum(cnts_ref[...], 1.0)
        o_ref[...] = jnp.dot(pooled, wl_ref[...],
                             preferred_element_type=jnp.float32) + bl_ref[...]


def _layer3_pool(a2, h2, wr2, wo2, b, segs, wl, bl):
    return pl.pallas_call(
        _layer3_pool_body,
        grid=(NP // BM,),
        in_specs=[
            pl.BlockSpec((2, BM, DH), lambda i: (0, i, 0)),
            pl.BlockSpec((2, BM, DH), lambda i: (0, i, 0)),
            pl.BlockSpec((2, DH, D), lambda i: (0, 0, 0)),
            pl.BlockSpec((2, DH, D), lambda i: (0, 0, 0)),
            pl.BlockSpec((1, D), lambda i: (0, 0)),
            pl.BlockSpec((BM,), lambda i: (i,)),
            pl.BlockSpec((D, D), lambda i: (0, 0)),
            pl.BlockSpec((1, D), lambda i: (0, 0)),
        ],
        out_specs=pl.BlockSpec((G, D), lambda i: (0, 0)),
        out_shape=jax.ShapeDtypeStruct((G, D), jnp.float32),
        scratch_shapes=[
            pltpu.VMEM((G, D), jnp.float32),
            pltpu.VMEM((G, D), jnp.float32),
        ],
    )(a2, h2, wr2, wo2, b, segs, wl, bl)


def _split_w(w):
    return jnp.stack([w[:DH], w[DH:]])


def kernel(x, edge_index, batch, bn_gamma, bn_beta,
           W1_rel, W1_root, b1, W2_rel, W2_root, b2,
           W3_rel, W3_root, b3, W_lin, b_lin):
    eps = 1e-5
    gv = (bn_gamma * (1.0 / math.sqrt(1.0 + eps)))[None, :]
    bv = bn_beta[None, :]

    xp = jnp.pad(x, ((0, NP - N), (0, 0)))
    src = jnp.pad(edge_index[0], (0, EP - E)).reshape(NSC, NCH, CHUNK)
    pad_dst = N + (jnp.arange(EP - E, dtype=jnp.int32) % (NP - N))
    dst = jnp.concatenate([edge_index[1], pad_dst]).reshape(NSC, NCH, CHUNK)
    segs = jnp.pad(batch, (0, NP - N), constant_values=G).astype(jnp.int32)
    wl = jnp.pad(W_lin, ((0, 0), (0, D - C)))
    bl = jnp.pad(b_lin, (0, D - C))[None, :]

    h2 = _bn(xp, gv, bv)

    for (wr, wo, b, relu) in (
        (W1_rel, W1_root, b1, True),
        (W2_rel, W2_root, b2, True),
    ):
        a2 = _sc_agg(h2, src, dst)
        h2 = _layer(a2, h2, _split_w(wr), _split_w(wo), b[None, :], relu)

    a2 = _sc_agg(h2, src, dst)
    out = _layer3_pool(a2, h2, _split_w(W3_rel), _split_w(W3_root),
                       b3[None, :], segs, wl, bl)
    return out[:, :C]


# speedup vs baseline: 1.2423x; 1.1195x over previous
import functools
import math

import jax
import jax.numpy as jnp
from jax import lax
from jax.experimental import pallas as pl
from jax.experimental.pallas import tpu as pltpu
from jax.experimental.pallas import tpu_sc as plsc

N = 10000
E = 320000
D = 128
DH = D // 2
G = 64
C = 10

NP = 10240
NSC = 16
CHUNK = 128
NCH = 160
NPH = 4
CPP = NCH // NPH
EP = NSC * NCH * CHUNK
ROWS_PER_SUB = NP // NSC
TAB_PER_SUB = NP // NSC
ZR = 16
NZB = ROWS_PER_SUB // ZR


def _sc_agg_body(ha_hbm, hb_hbm, srcs_hbm, dsts_hbm, out_hbm,
                 src_v, dst_v, rows_v, rows1_v, rows2_v, rows3_v, zero_v,
                 htab_sh, acc_sh, semg0, semg1, semg2, semg3,
                 sems0, sems1, sems2, sems3):
    c = lax.axis_index("c")
    s = lax.axis_index("s")

    @pl.when(c == 0)
    def _():
        pltpu.sync_copy(ha_hbm.at[pl.ds(s * TAB_PER_SUB, TAB_PER_SUB)],
                        htab_sh.at[pl.ds(s * TAB_PER_SUB, TAB_PER_SUB)])

    @pl.when(c == 1)
    def _():
        pltpu.sync_copy(hb_hbm.at[pl.ds(s * TAB_PER_SUB, TAB_PER_SUB)],
                        htab_sh.at[pl.ds(s * TAB_PER_SUB, TAB_PER_SUB)])

    def zrow(i, carry):
        for l in range(DH // 16):
            zero_v[i, pl.ds(l * 16, 16)] = jnp.zeros((16,), jnp.float32)
        return carry
    lax.fori_loop(0, ZR, zrow, 0)

    def zblk(j, carry):
        pltpu.sync_copy(zero_v, acc_sh.at[pl.ds(s * ROWS_PER_SUB + j * ZR, ZR)])
        return carry
    lax.fori_loop(0, NZB, zblk, 0)

    plsc.subcore_barrier()

    bufs = (rows_v, rows1_v, rows2_v, rows3_v)
    semg = (semg0, semg1, semg2, semg3)
    sems = (sems0, sems1, sems2, sems3)

    def gat(j, b):
        return pltpu.make_async_copy(htab_sh.at[src_v.at[j]], bufs[b], semg[b])

    def sca(j, b):
        return pltpu.make_async_copy(bufs[b], acc_sh.at[dst_v.at[j]], sems[b])

    for p in range(NPH):
        pltpu.sync_copy(srcs_hbm.at[s].at[pl.ds(p * CPP, CPP)], src_v)
        pltpu.sync_copy(dsts_hbm.at[s].at[pl.ds(p * CPP, CPP)], dst_v)

        gat(0, 0).start()
        gat(1, 1).start()

        def body(jj, carry):
            for b in range(4):
                j = 4 * jj + b
                gat(j, b).wait()
                pltpu.async_copy(bufs[b], acc_sh.at[dst_v.at[j]],
                                 sems[b], add=True)
                bq = (b + 2) % 4
                if b < 2:
                    @pl.when(jj > 0)
                    def _():
                        sca(j - 2, bq).wait()
                    gat(j + 2, bq).start()
                else:
                    sca(j - 2, bq).wait()

                    @pl.when(jj < CPP // 4 - 1)
                    def _():
                        gat(j + 2, bq).start()
            return carry
        lax.fori_loop(0, CPP // 4, body, 0)

        sca(CPP - 2, 2).wait()
        sca(CPP - 1, 3).wait()
    plsc.subcore_barrier()

    pltpu.sync_copy(acc_sh.at[pl.ds(s * ROWS_PER_SUB, ROWS_PER_SUB)],
                    out_hbm.at[c].at[pl.ds(s * ROWS_PER_SUB, ROWS_PER_SUB)])


@functools.cache
def _sc_agg_call():
    return pl.kernel(
        _sc_agg_body,
        out_type=jax.ShapeDtypeStruct((2, NP, DH), jnp.float32),
        mesh=plsc.VectorSubcoreMesh(core_axis_name="c", subcore_axis_name="s"),
        scratch_types=[
            pltpu.VMEM((CPP, CHUNK), jnp.int32),
            pltpu.VMEM((CPP, CHUNK), jnp.int32),
            pltpu.VMEM((CHUNK, DH), jnp.float32),
            pltpu.VMEM((CHUNK, DH), jnp.float32),
            pltpu.VMEM((CHUNK, DH), jnp.float32),
            pltpu.VMEM((CHUNK, DH), jnp.float32),
            pltpu.VMEM((ZR, DH), jnp.float32),
            pltpu.VMEM_SHARED((NP, DH), jnp.float32),
            pltpu.VMEM_SHARED((NP, DH), jnp.float32),
            pltpu.SemaphoreType.DMA,
            pltpu.SemaphoreType.DMA,
            pltpu.SemaphoreType.DMA,
            pltpu.SemaphoreType.DMA,
            pltpu.SemaphoreType.DMA,
            pltpu.SemaphoreType.DMA,
            pltpu.SemaphoreType.DMA,
            pltpu.SemaphoreType.DMA,
        ],
        compiler_params=pltpu.CompilerParams(use_tc_tiling_on_sc=False),
    )


def _sc_agg(h2, src, dst):
    return _sc_agg_call()(h2[0], h2[1], src, dst)


BM = 1024


def _bn_body(x_ref, g_ref, b_ref, o_ref):
    y = x_ref[...] * g_ref[...] + b_ref[...]
    o_ref[0] = y[:, :DH]
    o_ref[1] = y[:, DH:]


def _bn(x, gv, bv):
    return pl.pallas_call(
        _bn_body,
        grid=(NP // BM,),
        in_specs=[
            pl.BlockSpec((BM, D), lambda i: (i, 0)),
            pl.BlockSpec((1, D), lambda i: (0, 0)),
            pl.BlockSpec((1, D), lambda i: (0, 0)),
        ],
        out_specs=pl.BlockSpec((2, BM, DH), lambda i: (0, i, 0)),
        out_shape=jax.ShapeDtypeStruct((2, NP, DH), jnp.float32),
    )(x, gv, bv)


def _layer_body(a_ref, h_ref, wr_ref, wo_ref, b_ref, o_ref, *, relu):
    acc = jnp.dot(a_ref[0], wr_ref[0], preferred_element_type=jnp.float32)
    acc = acc + jnp.dot(a_ref[1], wr_ref[1], preferred_element_type=jnp.float32)
    acc = acc + jnp.dot(h_ref[0], wo_ref[0], preferred_element_type=jnp.float32)
    acc = acc + jnp.dot(h_ref[1], wo_ref[1], preferred_element_type=jnp.float32)
    acc = acc + b_ref[...]
    if relu:
        acc = jnp.maximum(acc, 0.0)
    o_ref[0] = acc[:, :DH]
    o_ref[1] = acc[:, DH:]


def _layer(a2, h2, wr2, wo2, b, relu):
    return pl.pallas_call(
        functools.partial(_layer_body, relu=relu),
        grid=(NP // BM,),
        in_specs=[
            pl.BlockSpec((2, BM, DH), lambda i: (0, i, 0)),
            pl.BlockSpec((2, BM, DH), lambda i: (0, i, 0)),
            pl.BlockSpec((2, DH, D), lambda i: (0, 0, 0)),
            pl.BlockSpec((2, DH, D), lambda i: (0, 0, 0)),
            pl.BlockSpec((1, D), lambda i: (0, 0)),
        ],
        out_specs=pl.BlockSpec((2, BM, DH), lambda i: (0, i, 0)),
        out_shape=jax.ShapeDtypeStruct((2, NP, DH), jnp.float32),
    )(a2, h2, wr2, wo2, b)


def _layer3_pool_body(a_ref, h_ref, wr_ref, wo_ref, b_ref, seg_ref,
                      wl_ref, bl_ref, o_ref, sums_ref, cnts_ref):
    i = pl.program_id(0)

    @pl.when(i == 0)
    def _():
        sums_ref[...] = jnp.zeros_like(sums_ref)
        cnts_ref[...] = jnp.zeros_like(cnts_ref)

    acc = jnp.dot(a_ref[0], wr_ref[0], preferred_element_type=jnp.float32)
    acc = acc + jnp.dot(a_ref[1], wr_ref[1], preferred_element_type=jnp.float32)
    acc = acc + jnp.dot(h_ref[0], wo_ref[0], preferred_element_type=jnp.float32)
    acc = acc + jnp.dot(h_ref[1], wo_ref[1], preferred_element_type=jnp.float32)
    acc = acc + b_ref[...]

    seg = seg_ref[...]
    onehot = jnp.where(
        seg[:, None] == lax.broadcasted_iota(jnp.int32, (1, G), 1),
        1.0, 0.0).astype(jnp.float32)
    sums_ref[...] += lax.dot_general(onehot, acc, (((0,), (0,)), ((), ())),
                                     preferred_element_type=jnp.float32)
    cnts_ref[...] += jnp.sum(onehot, axis=0)[:, None]

    @pl.when(i == pl.num_programs(0) - 1)
    def _():
        pooled = sums_ref[...] / jnp.maximum(cnts_ref[...], 1.0)
        o_ref[...] = jnp.dot(pooled, wl_ref[...],
                             preferred_element_type=jnp.float32) + bl_ref[...]


def _layer3_pool(a2, h2, wr2, wo2, b, segs, wl, bl):
    return pl.pallas_call(
        _layer3_pool_body,
        grid=(NP // BM,),
        in_specs=[
            pl.BlockSpec((2, BM, DH), lambda i: (0, i, 0)),
            pl.BlockSpec((2, BM, DH), lambda i: (0, i, 0)),
            pl.BlockSpec((2, DH, D), lambda i: (0, 0, 0)),
            pl.BlockSpec((2, DH, D), lambda i: (0, 0, 0)),
            pl.BlockSpec((1, D), lambda i: (0, 0)),
            pl.BlockSpec((BM,), lambda i: (i,)),
            pl.BlockSpec((D, D), lambda i: (0, 0)),
            pl.BlockSpec((1, D), lambda i: (0, 0)),
        ],
        out_specs=pl.BlockSpec((G, D), lambda i: (0, 0)),
        out_shape=jax.ShapeDtypeStruct((G, D), jnp.float32),
        scratch_shapes=[
            pltpu.VMEM((G, D), jnp.float32),
            pltpu.VMEM((G, D), jnp.float32),
        ],
    )(a2, h2, wr2, wo2, b, segs, wl, bl)


def _split_w(w):
    return jnp.stack([w[:DH], w[DH:]])


def kernel(x, edge_index, batch, bn_gamma, bn_beta,
           W1_rel, W1_root, b1, W2_rel, W2_root, b2,
           W3_rel, W3_root, b3, W_lin, b_lin):
    eps = 1e-5
    gv = (bn_gamma * (1.0 / math.sqrt(1.0 + eps)))[None, :]
    bv = bn_beta[None, :]

    xp = jnp.pad(x, ((0, NP - N), (0, 0)))
    src = jnp.pad(edge_index[0], (0, EP - E)).reshape(NSC, NCH, CHUNK)
    pad_dst = N + (jnp.arange(EP - E, dtype=jnp.int32) % (NP - N))
    dst = jnp.concatenate([edge_index[1], pad_dst]).reshape(NSC, NCH, CHUNK)
    segs = jnp.pad(batch, (0, NP - N), constant_values=G).astype(jnp.int32)
    wl = jnp.pad(W_lin, ((0, 0), (0, D - C)))
    bl = jnp.pad(b_lin, (0, D - C))[None, :]

    h2 = _bn(xp, gv, bv)

    for (wr, wo, b, relu) in (
        (W1_rel, W1_root, b1, True),
        (W2_rel, W2_root, b2, True),
    ):
        a2 = _sc_agg(h2, src, dst)
        h2 = _layer(a2, h2, _split_w(wr), _split_w(wo), b[None, :], relu)

    a2 = _sc_agg(h2, src, dst)
    out = _layer3_pool(a2, h2, _split_w(W3_rel), _split_w(W3_root),
                       b3[None, :], segs, wl, bl)
    return out[:, :C]


# speedup vs baseline: 1.2549x; 1.0102x over previous
import functools
import math

import jax
import jax.numpy as jnp
from jax import lax
from jax.experimental import pallas as pl
from jax.experimental.pallas import tpu as pltpu
from jax.experimental.pallas import tpu_sc as plsc

N = 10000
E = 320000
D = 128
DH = D // 2
G = 64
C = 10

NP = 10240
NSC = 16
CHUNK = 128
NCH = 160
NPH = 4
CPP = NCH // NPH
EP = NSC * NCH * CHUNK
ROWS_PER_SUB = NP // NSC
TAB_PER_SUB = NP // NSC
ZR = 16
NZB = ROWS_PER_SUB // ZR


def _sc_agg_body(ha_hbm, hb_hbm, srcs_hbm, dsts_hbm, out_hbm,
                 src_v, dst_v, rows_v, rows1_v, rows2_v, rows3_v, zero_v,
                 htab_sh, acc_sh, semg0, semg1, semg2, semg3,
                 sems0, sems1, sems2, sems3):
    c = lax.axis_index("c")
    s = lax.axis_index("s")

    @pl.when(c == 0)
    def _():
        pltpu.sync_copy(ha_hbm.at[pl.ds(s * TAB_PER_SUB, TAB_PER_SUB)],
                        htab_sh.at[pl.ds(s * TAB_PER_SUB, TAB_PER_SUB)])

    @pl.when(c == 1)
    def _():
        pltpu.sync_copy(hb_hbm.at[pl.ds(s * TAB_PER_SUB, TAB_PER_SUB)],
                        htab_sh.at[pl.ds(s * TAB_PER_SUB, TAB_PER_SUB)])

    def zrow(i, carry):
        for l in range(DH // 16):
            zero_v[i, pl.ds(l * 16, 16)] = jnp.zeros((16,), jnp.float32)
        return carry
    lax.fori_loop(0, ZR, zrow, 0)

    def zblk(j, carry):
        pltpu.make_async_copy(
            zero_v, acc_sh.at[pl.ds(s * ROWS_PER_SUB + j * ZR, ZR)],
            semg0).start()
        return carry
    lax.fori_loop(0, NZB, zblk, 0)

    def zdrain(j, carry):
        pltpu.make_async_copy(
            zero_v, acc_sh.at[pl.ds(s * ROWS_PER_SUB + j * ZR, ZR)],
            semg0).wait()
        return carry
    lax.fori_loop(0, NZB, zdrain, 0)

    plsc.subcore_barrier()

    bufs = (rows_v, rows1_v, rows2_v, rows3_v)
    semg = (semg0, semg1, semg2, semg3)
    sems = (sems0, sems1, sems2, sems3)

    def gat(j, b):
        return pltpu.make_async_copy(htab_sh.at[src_v.at[j]], bufs[b], semg[b])

    def sca(j, b):
        return pltpu.make_async_copy(bufs[b], acc_sh.at[dst_v.at[j]], sems[b])

    for p in range(NPH):
        pltpu.sync_copy(srcs_hbm.at[s].at[pl.ds(p * CPP, CPP)], src_v)
        pltpu.sync_copy(dsts_hbm.at[s].at[pl.ds(p * CPP, CPP)], dst_v)

        gat(0, 0).start()
        gat(1, 1).start()

        def body(jj, carry):
            for b in range(4):
                j = 4 * jj + b
                gat(j, b).wait()
                pltpu.async_copy(bufs[b], acc_sh.at[dst_v.at[j]],
                                 sems[b], add=True)
                bq = (b + 2) % 4
                if b < 2:
                    @pl.when(jj > 0)
                    def _():
                        sca(j - 2, bq).wait()
                    gat(j + 2, bq).start()
                else:
                    sca(j - 2, bq).wait()

                    @pl.when(jj < CPP // 4 - 1)
                    def _():
                        gat(j + 2, bq).start()
            return carry
        lax.fori_loop(0, CPP // 4, body, 0)

        sca(CPP - 2, 2).wait()
        sca(CPP - 1, 3).wait()
    plsc.subcore_barrier()

    pltpu.sync_copy(acc_sh.at[pl.ds(s * ROWS_PER_SUB, ROWS_PER_SUB)],
                    out_hbm.at[c].at[pl.ds(s * ROWS_PER_SUB, ROWS_PER_SUB)])


@functools.cache
def _sc_agg_call():
    return pl.kernel(
        _sc_agg_body,
        out_type=jax.ShapeDtypeStruct((2, NP, DH), jnp.float32),
        mesh=plsc.VectorSubcoreMesh(core_axis_name="c", subcore_axis_name="s"),
        scratch_types=[
            pltpu.VMEM((CPP, CHUNK), jnp.int32),
            pltpu.VMEM((CPP, CHUNK), jnp.int32),
            pltpu.VMEM((CHUNK, DH), jnp.float32),
            pltpu.VMEM((CHUNK, DH), jnp.float32),
            pltpu.VMEM((CHUNK, DH), jnp.float32),
            pltpu.VMEM((CHUNK, DH), jnp.float32),
            pltpu.VMEM((ZR, DH), jnp.float32),
            pltpu.VMEM_SHARED((NP, DH), jnp.float32),
            pltpu.VMEM_SHARED((NP, DH), jnp.float32),
            pltpu.SemaphoreType.DMA,
            pltpu.SemaphoreType.DMA,
            pltpu.SemaphoreType.DMA,
            pltpu.SemaphoreType.DMA,
            pltpu.SemaphoreType.DMA,
            pltpu.SemaphoreType.DMA,
            pltpu.SemaphoreType.DMA,
            pltpu.SemaphoreType.DMA,
        ],
        compiler_params=pltpu.CompilerParams(use_tc_tiling_on_sc=False),
    )


def _sc_agg(h2, src, dst):
    return _sc_agg_call()(h2[0], h2[1], src, dst)


BM = 1024


def _bn_body(x_ref, g_ref, b_ref, o_ref):
    y = x_ref[...] * g_ref[...] + b_ref[...]
    o_ref[0] = y[:, :DH]
    o_ref[1] = y[:, DH:]


def _bn(x, gv, bv):
    return pl.pallas_call(
        _bn_body,
        grid=(NP // BM,),
        in_specs=[
            pl.BlockSpec((BM, D), lambda i: (i, 0)),
            pl.BlockSpec((1, D), lambda i: (0, 0)),
            pl.BlockSpec((1, D), lambda i: (0, 0)),
        ],
        out_specs=pl.BlockSpec((2, BM, DH), lambda i: (0, i, 0)),
        out_shape=jax.ShapeDtypeStruct((2, NP, DH), jnp.float32),
    )(x, gv, bv)


def _layer_body(a_ref, h_ref, wr_ref, wo_ref, b_ref, o_ref, *, relu):
    acc = jnp.dot(a_ref[0], wr_ref[0], preferred_element_type=jnp.float32)
    acc = acc + jnp.dot(a_ref[1], wr_ref[1], preferred_element_type=jnp.float32)
    acc = acc + jnp.dot(h_ref[0], wo_ref[0], preferred_element_type=jnp.float32)
    acc = acc + jnp.dot(h_ref[1], wo_ref[1], preferred_element_type=jnp.float32)
    acc = acc + b_ref[...]
    if relu:
        acc = jnp.maximum(acc, 0.0)
    o_ref[0] = acc[:, :DH]
    o_ref[1] = acc[:, DH:]


def _layer(a2, h2, wr2, wo2, b, relu):
    return pl.pallas_call(
        functools.partial(_layer_body, relu=relu),
        grid=(NP // BM,),
        in_specs=[
            pl.BlockSpec((2, BM, DH), lambda i: (0, i, 0)),
            pl.BlockSpec((2, BM, DH), lambda i: (0, i, 0)),
            pl.BlockSpec((2, DH, D), lambda i: (0, 0, 0)),
            pl.BlockSpec((2, DH, D), lambda i: (0, 0, 0)),
            pl.BlockSpec((1, D), lambda i: (0, 0)),
        ],
        out_specs=pl.BlockSpec((2, BM, DH), lambda i: (0, i, 0)),
        out_shape=jax.ShapeDtypeStruct((2, NP, DH), jnp.float32),
    )(a2, h2, wr2, wo2, b)


def _layer3_pool_body(a_ref, h_ref, wr_ref, wo_ref, b_ref, seg_ref,
                      wl_ref, bl_ref, o_ref, sums_ref, cnts_ref):
    i = pl.program_id(0)

    @pl.when(i == 0)
    def _():
        sums_ref[...] = jnp.zeros_like(sums_ref)
        cnts_ref[...] = jnp.zeros_like(cnts_ref)

    acc = jnp.dot(a_ref[0], wr_ref[0], preferred_element_type=jnp.float32)
    acc = acc + jnp.dot(a_ref[1], wr_ref[1], preferred_element_type=jnp.float32)
    acc = acc + jnp.dot(h_ref[0], wo_ref[0], preferred_element_type=jnp.float32)
    acc = acc + jnp.dot(h_ref[1], wo_ref[1], preferred_element_type=jnp.float32)
    acc = acc + b_ref[...]

    seg = seg_ref[...]
    onehot = jnp.where(
        seg[:, None] == lax.broadcasted_iota(jnp.int32, (1, G), 1),
        1.0, 0.0).astype(jnp.float32)
    sums_ref[...] += lax.dot_general(onehot, acc, (((0,), (0,)), ((), ())),
                                     preferred_element_type=jnp.float32)
    cnts_ref[...] += jnp.sum(onehot, axis=0)[:, None]

    @pl.when(i == pl.num_programs(0) - 1)
    def _():
        pooled = sums_ref[...] / jnp.maximum(cnts_ref[...], 1.0)
        o_ref[...] = jnp.dot(pooled, wl_ref[...],
                             preferred_element_type=jnp.float32) + bl_ref[...]


def _layer3_pool(a2, h2, wr2, wo2, b, segs, wl, bl):
    return pl.pallas_call(
        _layer3_pool_body,
        grid=(NP // BM,),
        in_specs=[
            pl.BlockSpec((2, BM, DH), lambda i: (0, i, 0)),
            pl.BlockSpec((2, BM, DH), lambda i: (0, i, 0)),
            pl.BlockSpec((2, DH, D), lambda i: (0, 0, 0)),
            pl.BlockSpec((2, DH, D), lambda i: (0, 0, 0)),
            pl.BlockSpec((1, D), lambda i: (0, 0)),
            pl.BlockSpec((BM,), lambda i: (i,)),
            pl.BlockSpec((D, D), lambda i: (0, 0)),
            pl.BlockSpec((1, D), lambda i: (0, 0)),
        ],
        out_specs=pl.BlockSpec((G, D), lambda i: (0, 0)),
        out_shape=jax.ShapeDtypeStruct((G, D), jnp.float32),
        scratch_shapes=[
            pltpu.VMEM((G, D), jnp.float32),
            pltpu.VMEM((G, D), jnp.float32),
        ],
    )(a2, h2, wr2, wo2, b, segs, wl, bl)


def _split_w(w):
    return jnp.stack([w[:DH], w[DH:]])


def kernel(x, edge_index, batch, bn_gamma, bn_beta,
           W1_rel, W1_root, b1, W2_rel, W2_root, b2,
           W3_rel, W3_root, b3, W_lin, b_lin):
    eps = 1e-5
    gv = (bn_gamma * (1.0 / math.sqrt(1.0 + eps)))[None, :]
    bv = bn_beta[None, :]

    xp = jnp.pad(x, ((0, NP - N), (0, 0)))
    src = jnp.pad(edge_index[0], (0, EP - E)).reshape(NSC, NCH, CHUNK)
    pad_dst = N + (jnp.arange(EP - E, dtype=jnp.int32) % (NP - N))
    dst = jnp.concatenate([edge_index[1], pad_dst]).reshape(NSC, NCH, CHUNK)
    segs = jnp.pad(batch, (0, NP - N), constant_values=G).astype(jnp.int32)
    wl = jnp.pad(W_lin, ((0, 0), (0, D - C)))
    bl = jnp.pad(b_lin, (0, D - C))[None, :]

    h2 = _bn(xp, gv, bv)

    for (wr, wo, b, relu) in (
        (W1_rel, W1_root, b1, True),
        (W2_rel, W2_root, b2, True),
    ):
        a2 = _sc_agg(h2, src, dst)
        h2 = _layer(a2, h2, _split_w(wr), _split_w(wo), b[None, :], relu)

    a2 = _sc_agg(h2, src, dst)
    out = _layer3_pool(a2, h2, _split_w(W3_rel), _split_w(W3_root),
                       b3[None, :], segs, wl, bl)
    return out[:, :C]
